# baseline (speedup 1.0000x reference)
import functools

import jax
import jax.numpy as jnp
from jax import lax
from jax.experimental import pallas as pl
from jax.experimental.pallas import tpu as pltpu
from jax.experimental.pallas import tpu_sc as plsc

_NUM_CORES = 2
_NUM_SUBCORES = 16
_NUM_WORKERS = _NUM_CORES * _NUM_SUBCORES
_LANES = 16
_CHUNK = 128


def _gather_body(idx_hbm, wt_hbm, out_hbm, idx_vmem, addr, data, sem):
    dim, bpw = addr.shape
    vocab = wt_hbm.shape[0] // dim
    wid = lax.axis_index("s") * _NUM_CORES + lax.axis_index("c")
    base = wid * bpw

    pltpu.sync_copy(idx_hbm.at[pl.ds(base, bpw)], idx_vmem)

    def build_row(d, carry):
        for cj in range(bpw // _LANES):
            vec = idx_vmem[pl.ds(cj * _LANES, _LANES)]
            addr[d, pl.ds(cj * _LANES, _LANES)] = vec + d * vocab
        return carry

    lax.fori_loop(0, dim, build_row, 0)

    def fire_row(d, carry):
        for q in range(bpw // _CHUNK):
            pltpu.async_copy(
                wt_hbm.at[addr.at[d, pl.ds(q * _CHUNK, _CHUNK)]],
                data.at[d, pl.ds(q * _CHUNK, _CHUNK)],
                sem,
            )
        return carry

    lax.fori_loop(0, dim, fire_row, 0)

    pltpu.make_async_copy(wt_hbm.at[pl.ds(0, dim * bpw)], data, sem).wait()

    pltpu.sync_copy(data, out_hbm.at[:, pl.ds(base, bpw)])


@jax.jit
def kernel(input_, weight):
    batch = input_.shape[0]
    dim = weight.shape[1]
    bpw = batch // _NUM_WORKERS

    wt_flat = weight.T.reshape(-1)
    mesh = plsc.VectorSubcoreMesh(
        core_axis_name="c",
        subcore_axis_name="s",
        num_cores=_NUM_CORES,
        num_subcores=_NUM_SUBCORES,
    )
    run = pl.kernel(
        _gather_body,
        out_type=jax.ShapeDtypeStruct((dim, batch), weight.dtype),
        mesh=mesh,
        scratch_types=[
            pltpu.VMEM((bpw,), jnp.int32),
            pltpu.VMEM((dim, bpw), jnp.int32),
            pltpu.VMEM((dim, bpw), weight.dtype),
            pltpu.SemaphoreType.DMA,
        ],
        compiler_params=pltpu.CompilerParams(use_tc_tiling_on_sc=False),
    )
    return run(input_, wt_flat).T


# speedup vs baseline: 7.8905x; 7.8905x over previous
import functools

import jax
import jax.numpy as jnp
from jax import lax
from jax.experimental import pallas as pl
from jax.experimental.pallas import tpu as pltpu
from jax.experimental.pallas import tpu_sc as plsc

_NUM_CORES = 2
_NUM_SUBCORES = 16
_NUM_WORKERS = _NUM_CORES * _NUM_SUBCORES
_LANES = 16
_CHUNK = 128


def _gather_body(idx_hbm, w2_hbm, out_hbm, idx_v, pairs, cols, rows, stage, sem):
    bpw = idx_v.shape[0]
    dim = stage.shape[0]
    wid = lax.axis_index("s") * _NUM_CORES + lax.axis_index("c")
    base = wid * bpw

    pltpu.sync_copy(idx_hbm.at[pl.ds(base, bpw)], idx_v)

    def prep(cj, carry):
        vec = idx_v[pl.ds(cj * _LANES, _LANES)]
        pairs[pl.ds(cj * _LANES, _LANES)] = lax.shift_right_logical(vec, 1)
        cols[pl.ds(cj * _LANES, _LANES)] = lax.shift_left(
            lax.bitwise_and(vec, 1), 6
        )
        return carry

    lax.fori_loop(0, bpw // _LANES, prep, 0, unroll=8)

    for q in range(bpw // _CHUNK):
        pltpu.async_copy(
            w2_hbm.at[pairs.at[pl.ds(q * _CHUNK, _CHUNK)]],
            rows.at[pl.ds(q * _CHUNK, _CHUNK)],
            sem,
        )
    pltpu.make_async_copy(w2_hbm.at[pl.ds(0, bpw)], rows, sem).wait()

    lane = lax.iota(jnp.int32, _LANES)

    def select(cj, carry):
        j0 = cj * _LANES
        row_idx = j0 + lane
        col_base = cols[pl.ds(j0, _LANES)]
        for d in range(dim):
            vals = plsc.load_gather(rows, [row_idx, col_base + d])
            stage[d, pl.ds(j0, _LANES)] = vals
        return carry

    lax.fori_loop(0, bpw // _LANES, select, 0)

    pltpu.sync_copy(stage, out_hbm.at[:, pl.ds(base, bpw)])


@jax.jit
def kernel(input_, weight):
    batch = input_.shape[0]
    dim = weight.shape[1]
    bpw = batch // _NUM_WORKERS

    w2 = weight.reshape(weight.shape[0] // 2, 2 * dim)
    mesh = plsc.VectorSubcoreMesh(
        core_axis_name="c",
        subcore_axis_name="s",
        num_cores=_NUM_CORES,
        num_subcores=_NUM_SUBCORES,
    )
    run = pl.kernel(
        _gather_body,
        out_type=jax.ShapeDtypeStruct((dim, batch), weight.dtype),
        mesh=mesh,
        scratch_types=[
            pltpu.VMEM((bpw,), jnp.int32),
            pltpu.VMEM((bpw,), jnp.int32),
            pltpu.VMEM((bpw,), jnp.int32),
            pltpu.VMEM((bpw, 2 * dim), weight.dtype),
            pltpu.VMEM((dim, bpw), weight.dtype),
            pltpu.SemaphoreType.DMA,
        ],
        compiler_params=pltpu.CompilerParams(
            use_tc_tiling_on_sc=True, needs_layout_passes=False
        ),
    )
    return run(input_, w2).T


# speedup vs baseline: 13.6713x; 1.7326x over previous
import functools

import jax
import jax.numpy as jnp
from jax import lax
from jax.experimental import pallas as pl
from jax.experimental.pallas import tpu as pltpu
from jax.experimental.pallas import tpu_sc as plsc

_NUM_CORES = 2
_NUM_SUBCORES = 16
_NUM_WORKERS = _NUM_CORES * _NUM_SUBCORES
_LANES = 16
_CHUNK = 128

_TILE = 128
_PAIRS_PER_STEP = 16


def _transpose_body(x_ref, o_ref):
    for t in range(_PAIRS_PER_STEP):
        xe = x_ref[:, (2 * t) * _TILE : (2 * t + 1) * _TILE]
        xo = x_ref[:, (2 * t + 1) * _TILE : (2 * t + 2) * _TILE]
        o_ref[t * _TILE : (t + 1) * _TILE, :] = jnp.concatenate(
            [xe.T, xo.T], axis=1
        )


def _gather_body(idx_hbm, w2_hbm, out_hbm, idx_v, pairs, cols, rows, stage, sem):
    bpw = idx_v.shape[0]
    dim = stage.shape[0]
    wid = lax.axis_index("s") * _NUM_CORES + lax.axis_index("c")
    base = wid * bpw

    pltpu.sync_copy(idx_hbm.at[pl.ds(base, bpw)], idx_v)

    def prep(cj, carry):
        vec = idx_v[pl.ds(cj * _LANES, _LANES)]
        pairs[pl.ds(cj * _LANES, _LANES)] = lax.shift_left(
            lax.shift_right_logical(vec, 8), 7
        ) + lax.bitwise_and(vec, _TILE - 1)
        cols[pl.ds(cj * _LANES, _LANES)] = lax.shift_left(
            lax.bitwise_and(lax.shift_right_logical(vec, 7), 1), 6
        )
        return carry

    lax.fori_loop(0, bpw // _LANES, prep, 0, unroll=8)

    for q in range(bpw // _CHUNK):
        pltpu.async_copy(
            w2_hbm.at[pairs.at[pl.ds(q * _CHUNK, _CHUNK)]],
            rows.at[pl.ds(q * _CHUNK, _CHUNK)],
            sem,
        )
    pltpu.make_async_copy(w2_hbm.at[pl.ds(0, bpw)], rows, sem).wait()

    lane = lax.iota(jnp.int32, _LANES)

    def select(cj, carry):
        j0 = cj * _LANES
        row_idx = j0 + lane
        col_base = cols[pl.ds(j0, _LANES)]
        for d in range(dim):
            vals = plsc.load_gather(rows, [row_idx, col_base + d])
            stage[d, pl.ds(j0, _LANES)] = vals
        return carry

    lax.fori_loop(0, bpw // _LANES, select, 0)

    pltpu.sync_copy(stage, out_hbm.at[:, pl.ds(base, bpw)])


@jax.jit
def kernel(input_, weight):
    batch = input_.shape[0]
    vocab = weight.shape[0]
    dim = weight.shape[1]
    bpw = batch // _NUM_WORKERS

    cols_per_step = 2 * _PAIRS_PER_STEP * _TILE
    steps = -(-vocab // cols_per_step)
    n_pairs = steps * _PAIRS_PER_STEP * _TILE

    wt = weight.T
    w2 = pl.pallas_call(
        _transpose_body,
        grid=(steps,),
        in_specs=[
            pl.BlockSpec((dim, cols_per_step), lambda i: (0, i)),
        ],
        out_specs=pl.BlockSpec(
            (_PAIRS_PER_STEP * _TILE, 2 * dim), lambda i: (i, 0)
        ),
        out_shape=jax.ShapeDtypeStruct((n_pairs, 2 * dim), weight.dtype),
        compiler_params=pltpu.CompilerParams(
            dimension_semantics=("parallel",),
        ),
    )(wt)

    mesh = plsc.VectorSubcoreMesh(
        core_axis_name="c",
        subcore_axis_name="s",
        num_cores=_NUM_CORES,
        num_subcores=_NUM_SUBCORES,
    )
    run = pl.kernel(
        _gather_body,
        out_type=jax.ShapeDtypeStruct((dim, batch), weight.dtype),
        mesh=mesh,
        scratch_types=[
            pltpu.VMEM((bpw,), jnp.int32),
            pltpu.VMEM((bpw,), jnp.int32),
            pltpu.VMEM((bpw,), jnp.int32),
            pltpu.VMEM((bpw, 2 * dim), weight.dtype),
            pltpu.VMEM((dim, bpw), weight.dtype),
            pltpu.SemaphoreType.DMA,
        ],
        compiler_params=pltpu.CompilerParams(
            use_tc_tiling_on_sc=True, needs_layout_passes=False
        ),
    )
    return run(input_, w2).T


# speedup vs baseline: 13.7421x; 1.0052x over previous
import functools

import jax
import jax.numpy as jnp
from jax import lax
from jax.experimental import pallas as pl
from jax.experimental.pallas import tpu as pltpu
from jax.experimental.pallas import tpu_sc as plsc

_NUM_CORES = 2
_NUM_SUBCORES = 16
_NUM_WORKERS = _NUM_CORES * _NUM_SUBCORES
_LANES = 16
_CHUNK = 128

_TILE = 128
_PAIRS_PER_STEP = 16


def _transpose_body(x_ref, o_ref):
    dim = x_ref.shape[0]
    for t in range(_PAIRS_PER_STEP):
        xe = x_ref[:, (2 * t) * _TILE : (2 * t + 1) * _TILE]
        xo = x_ref[:, (2 * t + 1) * _TILE : (2 * t + 2) * _TILE]
        o_ref[t * _TILE : (t + 1) * _TILE, 0:dim] = xe.T
        o_ref[t * _TILE : (t + 1) * _TILE, dim : 2 * dim] = xo.T


def _gather_body(idx_hbm, w2_hbm, out_hbm, idx_v, pairs, cols, rows, stage, sem):
    bpw = idx_v.shape[0]
    dim = stage.shape[0]
    wid = lax.axis_index("s") * _NUM_CORES + lax.axis_index("c")
    base = wid * bpw

    pltpu.sync_copy(idx_hbm.at[pl.ds(base, bpw)], idx_v)

    def prep(cj, carry):
        vec = idx_v[pl.ds(cj * _LANES, _LANES)]
        pairs[pl.ds(cj * _LANES, _LANES)] = lax.shift_left(
            lax.shift_right_logical(vec, 8), 7
        ) + lax.bitwise_and(vec, _TILE - 1)
        cols[pl.ds(cj * _LANES, _LANES)] = lax.shift_left(
            lax.bitwise_and(lax.shift_right_logical(vec, 7), 1), 6
        )
        return carry

    lax.fori_loop(0, bpw // _LANES, prep, 0, unroll=8)

    for q in range(bpw // _CHUNK):
        pltpu.async_copy(
            w2_hbm.at[pairs.at[pl.ds(q * _CHUNK, _CHUNK)]],
            rows.at[pl.ds(q * _CHUNK, _CHUNK)],
            sem,
        )
    pltpu.make_async_copy(w2_hbm.at[pl.ds(0, bpw)], rows, sem).wait()

    lane = lax.iota(jnp.int32, _LANES)

    def select(cj, carry):
        j0 = cj * _LANES
        row_idx = j0 + lane
        col_base = cols[pl.ds(j0, _LANES)]
        for d in range(dim):
            vals = plsc.load_gather(rows, [row_idx, col_base + d])
            stage[d, pl.ds(j0, _LANES)] = vals
        return carry

    lax.fori_loop(0, bpw // _LANES, select, 0)

    pltpu.sync_copy(stage, out_hbm.at[:, pl.ds(base, bpw)])


@jax.jit
def kernel(input_, weight):
    batch = input_.shape[0]
    vocab = weight.shape[0]
    dim = weight.shape[1]
    bpw = batch // _NUM_WORKERS

    cols_per_step = 2 * _PAIRS_PER_STEP * _TILE
    steps = -(-vocab // cols_per_step)
    n_pairs = steps * _PAIRS_PER_STEP * _TILE

    wt = weight.T
    w2 = pl.pallas_call(
        _transpose_body,
        grid=(steps,),
        in_specs=[
            pl.BlockSpec((dim, cols_per_step), lambda i: (0, i)),
        ],
        out_specs=pl.BlockSpec(
            (_PAIRS_PER_STEP * _TILE, 2 * dim), lambda i: (i, 0)
        ),
        out_shape=jax.ShapeDtypeStruct((n_pairs, 2 * dim), weight.dtype),
        compiler_params=pltpu.CompilerParams(
            dimension_semantics=("parallel",),
        ),
    )(wt)

    mesh = plsc.VectorSubcoreMesh(
        core_axis_name="c",
        subcore_axis_name="s",
        num_cores=_NUM_CORES,
        num_subcores=_NUM_SUBCORES,
    )
    run = pl.kernel(
        _gather_body,
        out_type=jax.ShapeDtypeStruct((dim, batch), weight.dtype),
        mesh=mesh,
        scratch_types=[
            pltpu.VMEM((bpw,), jnp.int32),
            pltpu.VMEM((bpw,), jnp.int32),
            pltpu.VMEM((bpw,), jnp.int32),
            pltpu.VMEM((bpw, 2 * dim), weight.dtype),
            pltpu.VMEM((dim, bpw), weight.dtype),
            pltpu.SemaphoreType.DMA,
        ],
        compiler_params=pltpu.CompilerParams(
            use_tc_tiling_on_sc=True, needs_layout_passes=False
        ),
    )
    return run(input_, w2).T


# speedup vs baseline: 18.8420x; 1.3711x over previous
import functools

import jax
import jax.numpy as jnp
from jax import lax
from jax.experimental import pallas as pl
from jax.experimental.pallas import tpu as pltpu
from jax.experimental.pallas import tpu_sc as plsc

_NUM_CORES = 2
_NUM_SUBCORES = 16
_NUM_WORKERS = _NUM_CORES * _NUM_SUBCORES
_LANES = 16
_CHUNK = 128

_TILE = 128
_PAIRS_PER_STEP = 64


def _transpose_body(x_ref, o_ref):
    dim = x_ref.shape[0]
    for t in range(_PAIRS_PER_STEP):
        xe = x_ref[:, (2 * t) * _TILE : (2 * t + 1) * _TILE]
        xo = x_ref[:, (2 * t + 1) * _TILE : (2 * t + 2) * _TILE]
        o_ref[t * _TILE : (t + 1) * _TILE, 0:dim] = xe.T
        o_ref[t * _TILE : (t + 1) * _TILE, dim : 2 * dim] = xo.T


def _gather_body(idx_hbm, w2_hbm, out_hbm, idx_v, pairs, cols, rows, stage, sem):
    bpw = idx_v.shape[0]
    dim = stage.shape[0]
    wid = lax.axis_index("s") * _NUM_CORES + lax.axis_index("c")
    base = wid * bpw

    pltpu.sync_copy(idx_hbm.at[pl.ds(base, bpw)], idx_v)

    def prep(cj, carry):
        vec = idx_v[pl.ds(cj * _LANES, _LANES)]
        pairs[pl.ds(cj * _LANES, _LANES)] = lax.shift_left(
            lax.shift_right_logical(vec, 8), 7
        ) + lax.bitwise_and(vec, _TILE - 1)
        cols[pl.ds(cj * _LANES, _LANES)] = lax.shift_left(
            lax.bitwise_and(lax.shift_right_logical(vec, 7), 1), 6
        )
        return carry

    lax.fori_loop(0, bpw // _LANES, prep, 0, unroll=8)

    for q in range(bpw // _CHUNK):
        pltpu.async_copy(
            w2_hbm.at[pairs.at[pl.ds(q * _CHUNK, _CHUNK)]],
            rows.at[pl.ds(q * _CHUNK, _CHUNK)],
            sem,
        )
    pltpu.make_async_copy(w2_hbm.at[pl.ds(0, bpw)], rows, sem).wait()

    lane = lax.iota(jnp.int32, _LANES)

    def select(cj, carry):
        j0 = cj * _LANES
        row_idx = j0 + lane
        col_base = cols[pl.ds(j0, _LANES)]
        for d in range(dim):
            vals = plsc.load_gather(rows, [row_idx, col_base + d])
            stage[d, pl.ds(j0, _LANES)] = vals
        return carry

    lax.fori_loop(0, bpw // _LANES, select, 0)

    pltpu.sync_copy(stage, out_hbm.at[:, pl.ds(base, bpw)])


@jax.jit
def kernel(input_, weight):
    batch = input_.shape[0]
    vocab = weight.shape[0]
    dim = weight.shape[1]
    bpw = batch // _NUM_WORKERS

    cols_per_step = 2 * _PAIRS_PER_STEP * _TILE
    steps = -(-vocab // cols_per_step)
    n_pairs = steps * _PAIRS_PER_STEP * _TILE

    wt = weight.T
    w2 = pl.pallas_call(
        _transpose_body,
        grid=(steps,),
        in_specs=[
            pl.BlockSpec((dim, cols_per_step), lambda i: (0, i)),
        ],
        out_specs=pl.BlockSpec(
            (_PAIRS_PER_STEP * _TILE, 2 * dim), lambda i: (i, 0)
        ),
        out_shape=jax.ShapeDtypeStruct((n_pairs, 2 * dim), weight.dtype),
        compiler_params=pltpu.CompilerParams(
            dimension_semantics=("parallel",),
            vmem_limit_bytes=100 * 1024 * 1024,
        ),
    )(wt)

    mesh = plsc.VectorSubcoreMesh(
        core_axis_name="c",
        subcore_axis_name="s",
        num_cores=_NUM_CORES,
        num_subcores=_NUM_SUBCORES,
    )
    run = pl.kernel(
        _gather_body,
        out_type=jax.ShapeDtypeStruct((dim, batch), weight.dtype),
        mesh=mesh,
        scratch_types=[
            pltpu.VMEM((bpw,), jnp.int32),
            pltpu.VMEM((bpw,), jnp.int32),
            pltpu.VMEM((bpw,), jnp.int32),
            pltpu.VMEM((bpw, 2 * dim), weight.dtype),
            pltpu.VMEM((dim, bpw), weight.dtype),
            pltpu.SemaphoreType.DMA,
        ],
        compiler_params=pltpu.CompilerParams(
            use_tc_tiling_on_sc=True, needs_layout_passes=False
        ),
    )
    return run(input_, w2).T


# speedup vs baseline: 19.9352x; 1.0580x over previous
import functools

import jax
import jax.numpy as jnp
from jax import lax
from jax.experimental import pallas as pl
from jax.experimental.pallas import tpu as pltpu
from jax.experimental.pallas import tpu_sc as plsc

_NUM_CORES = 2
_NUM_SUBCORES = 16
_NUM_WORKERS = _NUM_CORES * _NUM_SUBCORES
_LANES = 16
_CHUNK = 128

_TILE = 128
_PAIRS_PER_STEP = 128


def _transpose_body(x_ref, o_ref):
    dim = x_ref.shape[0]
    for t in range(_PAIRS_PER_STEP):
        xe = x_ref[:, (2 * t) * _TILE : (2 * t + 1) * _TILE]
        xo = x_ref[:, (2 * t + 1) * _TILE : (2 * t + 2) * _TILE]
        o_ref[t * _TILE : (t + 1) * _TILE, 0:dim] = xe.T
        o_ref[t * _TILE : (t + 1) * _TILE, dim : 2 * dim] = xo.T


def _gather_body(idx_hbm, w2_hbm, out_hbm, idx_v, pairs, cols, rows, stage, sem):
    bpw = idx_v.shape[0]
    dim = stage.shape[0]
    wid = lax.axis_index("s") * _NUM_CORES + lax.axis_index("c")
    base = wid * bpw

    pltpu.sync_copy(idx_hbm.at[pl.ds(base, bpw)], idx_v)

    def prep(cj, carry):
        vec = idx_v[pl.ds(cj * _LANES, _LANES)]
        pairs[pl.ds(cj * _LANES, _LANES)] = lax.shift_left(
            lax.shift_right_logical(vec, 8), 7
        ) + lax.bitwise_and(vec, _TILE - 1)
        cols[pl.ds(cj * _LANES, _LANES)] = lax.shift_left(
            lax.bitwise_and(lax.shift_right_logical(vec, 7), 1), 6
        )
        return carry

    lax.fori_loop(0, bpw // _LANES, prep, 0, unroll=8)

    for q in range(bpw // _CHUNK):
        pltpu.async_copy(
            w2_hbm.at[pairs.at[pl.ds(q * _CHUNK, _CHUNK)]],
            rows.at[pl.ds(q * _CHUNK, _CHUNK)],
            sem,
        )
    pltpu.make_async_copy(w2_hbm.at[pl.ds(0, bpw)], rows, sem).wait()

    lane = lax.iota(jnp.int32, _LANES)

    def select(cj, carry):
        j0 = cj * _LANES
        row_idx = j0 + lane
        col_base = cols[pl.ds(j0, _LANES)]
        for d in range(dim):
            vals = plsc.load_gather(rows, [row_idx, col_base + d])
            stage[d, pl.ds(j0, _LANES)] = vals
        return carry

    lax.fori_loop(0, bpw // _LANES, select, 0)

    pltpu.sync_copy(stage, out_hbm.at[:, pl.ds(base, bpw)])


@jax.jit
def kernel(input_, weight):
    batch = input_.shape[0]
    vocab = weight.shape[0]
    dim = weight.shape[1]
    bpw = batch // _NUM_WORKERS

    cols_per_step = 2 * _PAIRS_PER_STEP * _TILE
    steps = -(-vocab // cols_per_step)
    n_pairs = steps * _PAIRS_PER_STEP * _TILE

    wt = weight.T
    w2 = pl.pallas_call(
        _transpose_body,
        grid=(steps,),
        in_specs=[
            pl.BlockSpec((dim, cols_per_step), lambda i: (0, i)),
        ],
        out_specs=pl.BlockSpec(
            (_PAIRS_PER_STEP * _TILE, 2 * dim), lambda i: (i, 0)
        ),
        out_shape=jax.ShapeDtypeStruct((n_pairs, 2 * dim), weight.dtype),
        compiler_params=pltpu.CompilerParams(
            dimension_semantics=("parallel",),
            vmem_limit_bytes=100 * 1024 * 1024,
        ),
    )(wt)

    mesh = plsc.VectorSubcoreMesh(
        core_axis_name="c",
        subcore_axis_name="s",
        num_cores=_NUM_CORES,
        num_subcores=_NUM_SUBCORES,
    )
    run = pl.kernel(
        _gather_body,
        out_type=jax.ShapeDtypeStruct((dim, batch), weight.dtype),
        mesh=mesh,
        scratch_types=[
            pltpu.VMEM((bpw,), jnp.int32),
            pltpu.VMEM((bpw,), jnp.int32),
            pltpu.VMEM((bpw,), jnp.int32),
            pltpu.VMEM((bpw, 2 * dim), weight.dtype),
            pltpu.VMEM((dim, bpw), weight.dtype),
            pltpu.SemaphoreType.DMA,
        ],
        compiler_params=pltpu.CompilerParams(
            use_tc_tiling_on_sc=True, needs_layout_passes=False
        ),
    )
    return run(input_, w2).T
